# baseline (speedup 1.0000x reference)
import jax
import jax.numpy as jnp
from jax import lax
from jax.experimental import pallas as pl
from jax.experimental.pallas import tpu as pltpu

_EPS = 1e-5


def _round_up(x, m):
    return -(-x // m) * m


def _mlp3_body(x_ref, w1_ref, b1_ref, w2_ref, b2_ref, w3_ref, b3_ref, o_ref):
    half = x_ref.shape[0] // 2

    def run(sl):
        x = x_ref[sl, :].astype(jnp.bfloat16)
        h = jnp.dot(x, w1_ref[...], preferred_element_type=jnp.float32)
        h = jnp.maximum(h + b1_ref[...], 0.0).astype(jnp.bfloat16)
        h = jnp.dot(h, w2_ref[...], preferred_element_type=jnp.float32)
        h = jnp.maximum(h + b2_ref[...], 0.0).astype(jnp.bfloat16)
        o_ref[sl, :] = (jnp.dot(h, w3_ref[...],
                                preferred_element_type=jnp.float32)
                        + b3_ref[...]).astype(o_ref.dtype)

    run(pl.ds(0, half))
    run(pl.ds(half, half))


def kernel(x, w1, b1, g1, be1, m1, v1, w2, b2, g2, be2, m2, v2, w3, b3):
    s1 = g1 * lax.rsqrt(v1 + _EPS)
    w1f = (w1 * s1).astype(jnp.bfloat16)
    b1f = (b1 - m1) * s1 + be1
    s2 = g2 * lax.rsqrt(v2 + _EPS)
    w2f = (w2 * s2).astype(jnp.bfloat16)
    b2f = (b2 - m2) * s2 + be2

    B, dim_in = x.shape
    l = w1f.shape[1]
    dim_out = w3.shape[1]
    dim_out_p = max(128, _round_up(dim_out, 128))
    if dim_out_p != dim_out:
        w3 = jnp.pad(w3, ((0, 0), (0, dim_out_p - dim_out)))
        b3 = jnp.pad(b3, ((0, 0), (0, dim_out_p - dim_out)))
    w3b = w3.astype(jnp.bfloat16)

    TB = 512 if B >= 512 else max(8, _round_up(B, 8))
    B_pad = _round_up(B, TB)
    xb = x
    if B_pad != B:
        xb = jnp.pad(xb, ((0, B_pad - B), (0, 0)))
    grid = (B_pad // TB,)

    bf2, f4 = 2, 4
    footprint = (bf2 * (dim_in * l + l * l + l * dim_out_p)
                 + f4 * (2 * l + dim_out_p)
                 + 2 * (f4 * TB * dim_in + f4 * TB * dim_out_p)
                 + f4 * TB * l + bf2 * TB * l)
    vmem_limit = int(min(max(2 * footprint, 16 << 20), 48 << 20))

    const = lambda shape: pl.BlockSpec(shape, lambda i: (0, 0))
    out_p = pl.pallas_call(
        _mlp3_body,
        out_shape=jax.ShapeDtypeStruct((B_pad, dim_out_p), jnp.float32),
        grid=grid,
        in_specs=[
            pl.BlockSpec((TB, dim_in), lambda i: (i, 0)),
            const(w1f.shape), const(b1f.shape),
            const(w2f.shape), const(b2f.shape),
            const(w3b.shape), const(b3.shape),
        ],
        out_specs=pl.BlockSpec((TB, dim_out_p), lambda i: (i, 0)),
        compiler_params=pltpu.CompilerParams(
            dimension_semantics=("parallel",),
            vmem_limit_bytes=vmem_limit,
        ),
    )(xb, w1f, b1f, w2f, b2f, w3b, b3)

    return out_p[:B, :dim_out]


# speedup vs baseline: 1.1380x; 1.1380x over previous
import jax
import jax.numpy as jnp
from jax import lax
from jax.experimental import pallas as pl
from jax.experimental.pallas import tpu as pltpu

_EPS = 1e-5


def _round_up(x, m):
    return -(-x // m) * m


def _mlp3_body(x_ref, w1_ref, b1_ref, w2_ref, b2_ref, w3_ref, b3_ref, o_ref):
    x = x_ref[...].astype(jnp.bfloat16)
    h = jnp.dot(x, w1_ref[...], preferred_element_type=jnp.float32)
    h = jnp.maximum(h + b1_ref[...], 0.0).astype(jnp.bfloat16)
    h = jnp.dot(h, w2_ref[...], preferred_element_type=jnp.float32)
    h = jnp.maximum(h + b2_ref[...], 0.0).astype(jnp.bfloat16)
    o_ref[...] = (jnp.dot(h, w3_ref[...], preferred_element_type=jnp.float32)
                  + b3_ref[...]).astype(o_ref.dtype)


def kernel(x, w1, b1, g1, be1, m1, v1, w2, b2, g2, be2, m2, v2, w3, b3):
    s1 = g1 * lax.rsqrt(v1 + _EPS)
    w1f = (w1 * s1).astype(jnp.bfloat16)
    b1f = (b1 - m1) * s1 + be1
    s2 = g2 * lax.rsqrt(v2 + _EPS)
    w2f = (w2 * s2).astype(jnp.bfloat16)
    b2f = (b2 - m2) * s2 + be2

    B, dim_in = x.shape
    l = w1f.shape[1]
    dim_out = w3.shape[1]
    dim_out_p = max(128, _round_up(dim_out, 128))
    if dim_out_p != dim_out:
        w3 = jnp.pad(w3, ((0, 0), (0, dim_out_p - dim_out)))
        b3 = jnp.pad(b3, ((0, 0), (0, dim_out_p - dim_out)))
    w3b = w3.astype(jnp.bfloat16)

    TB = 1024 if B >= 1024 else max(8, _round_up(B, 8))
    B_pad = _round_up(B, TB)
    xb = x
    if B_pad != B:
        xb = jnp.pad(xb, ((0, B_pad - B), (0, 0)))
    grid = (B_pad // TB,)

    bf2, f4 = 2, 4
    footprint = (bf2 * (dim_in * l + l * l + l * dim_out_p)
                 + f4 * (2 * l + dim_out_p)
                 + 2 * (f4 * TB * dim_in + f4 * TB * dim_out_p)
                 + f4 * TB * l + bf2 * TB * l)
    vmem_limit = int(min(max(2 * footprint, 16 << 20), 48 << 20))

    const = lambda shape: pl.BlockSpec(shape, lambda i: (0, 0))
    out_p = pl.pallas_call(
        _mlp3_body,
        out_shape=jax.ShapeDtypeStruct((B_pad, dim_out_p), jnp.float32),
        grid=grid,
        in_specs=[
            pl.BlockSpec((TB, dim_in), lambda i: (i, 0)),
            const(w1f.shape), const(b1f.shape),
            const(w2f.shape), const(b2f.shape),
            const(w3b.shape), const(b3.shape),
        ],
        out_specs=pl.BlockSpec((TB, dim_out_p), lambda i: (i, 0)),
        compiler_params=pltpu.CompilerParams(
            dimension_semantics=("parallel",),
            vmem_limit_bytes=vmem_limit,
        ),
    )(xb, w1f, b1f, w2f, b2f, w3b, b3)

    return out_p[:B, :dim_out]


# speedup vs baseline: 1.1402x; 1.0020x over previous
import jax
import jax.numpy as jnp
from jax import lax
from jax.experimental import pallas as pl
from jax.experimental.pallas import tpu as pltpu

_EPS = 1e-5


def _round_up(x, m):
    return -(-x // m) * m


def _mlp3_body(x_ref, w1_ref, b1_ref, w2_ref, b2_ref, w3_ref, b3_ref, o_ref):
    x = x_ref[...].astype(jnp.bfloat16)
    h = jnp.dot(x, w1_ref[...], preferred_element_type=jnp.float32)
    h = jnp.maximum(h + b1_ref[...], 0.0).astype(jnp.bfloat16)
    h = jnp.dot(h, w2_ref[...], preferred_element_type=jnp.float32)
    h = jnp.maximum(h + b2_ref[...], 0.0).astype(jnp.bfloat16)
    o_ref[...] = (jnp.dot(h, w3_ref[...], preferred_element_type=jnp.float32)
                  + b3_ref[...]).astype(o_ref.dtype)


def kernel(x, w1, b1, g1, be1, m1, v1, w2, b2, g2, be2, m2, v2, w3, b3):
    s1 = g1 * lax.rsqrt(v1 + _EPS)
    w1f = (w1 * s1).astype(jnp.bfloat16)
    b1f = (b1 - m1) * s1 + be1
    s2 = g2 * lax.rsqrt(v2 + _EPS)
    w2f = (w2 * s2).astype(jnp.bfloat16)
    b2f = (b2 - m2) * s2 + be2

    B, dim_in = x.shape
    l = w1f.shape[1]
    dim_out = w3.shape[1]
    dim_out_p = max(128, _round_up(dim_out, 128))
    if dim_out_p != dim_out:
        w3 = jnp.pad(w3, ((0, 0), (0, dim_out_p - dim_out)))
        b3 = jnp.pad(b3, ((0, 0), (0, dim_out_p - dim_out)))
    w3b = w3.astype(jnp.bfloat16)

    TB = 2048 if B >= 2048 else max(8, _round_up(B, 8))
    B_pad = _round_up(B, TB)
    xb = x
    if B_pad != B:
        xb = jnp.pad(xb, ((0, B_pad - B), (0, 0)))
    grid = (B_pad // TB,)

    bf2, f4 = 2, 4
    footprint = (bf2 * (dim_in * l + l * l + l * dim_out_p)
                 + f4 * (2 * l + dim_out_p)
                 + 2 * (f4 * TB * dim_in + f4 * TB * dim_out_p)
                 + f4 * TB * l + bf2 * TB * l)
    vmem_limit = int(min(max(2 * footprint, 16 << 20), 48 << 20))

    const = lambda shape: pl.BlockSpec(shape, lambda i: (0, 0))
    out_p = pl.pallas_call(
        _mlp3_body,
        out_shape=jax.ShapeDtypeStruct((B_pad, dim_out_p), jnp.float32),
        grid=grid,
        in_specs=[
            pl.BlockSpec((TB, dim_in), lambda i: (i, 0)),
            const(w1f.shape), const(b1f.shape),
            const(w2f.shape), const(b2f.shape),
            const(w3b.shape), const(b3.shape),
        ],
        out_specs=pl.BlockSpec((TB, dim_out_p), lambda i: (i, 0)),
        compiler_params=pltpu.CompilerParams(
            dimension_semantics=("parallel",),
            vmem_limit_bytes=vmem_limit,
        ),
    )(xb, w1f, b1f, w2f, b2f, w3b, b3)

    return out_p[:B, :dim_out]
